# baseline (speedup 1.0000x reference)
import functools

import jax
import jax.numpy as jnp
from jax import lax
from jax.experimental import pallas as pl
from jax.experimental.pallas import tpu as pltpu
from jax.experimental.pallas import tpu_sc as plsc


_T = 256
_FT = 128


def _sc_gather_rows(table, idx, chunk=16):
    rows, d = idx.shape[0], table.shape[1]
    info = plsc.get_sparse_core_info()
    nw = info.num_cores * info.num_subcores
    per_w = rows // nw
    assert per_w % chunk == 0 and (per_w * (nw - 1)) % 8 == 0
    n_ch = per_w // chunk
    mesh = plsc.VectorSubcoreMesh(core_axis_name="c", subcore_axis_name="s")

    @functools.partial(
        pl.kernel,
        mesh=mesh,
        out_type=jax.ShapeDtypeStruct((rows, d), table.dtype),
        scratch_types=[
            pltpu.VMEM((chunk,), jnp.int32),
            pltpu.VMEM((chunk, d), table.dtype),
            pltpu.SemaphoreType.DMA,
        ],
    )
    def gather_k(table_hbm, idx_hbm, out_hbm, idx_v, rows_v, sem):
        wid = lax.axis_index("s") * info.num_cores + lax.axis_index("c")
        base = wid * per_w
        for c in range(n_ch):
            off = base + c * chunk
            pltpu.sync_copy(idx_hbm.at[pl.ds(off, chunk)], idx_v)
            pltpu.async_copy(table_hbm.at[idx_v], rows_v, sem).wait()
            pltpu.sync_copy(rows_v, out_hbm.at[pl.ds(off, chunk)])

    return gather_k(table, idx)


def _moe_body(n0_ref, x_ref, wgu, wuu, wdu, wgg, wug, wdg, y_ref, *, l, d, f_dim):
    f = pl.program_id(0)
    t = pl.program_id(1)
    n0 = n0_ref[0]
    row0 = t * _T
    x = x_ref[pl.ds(row0, _T), :]

    @pl.when(f == 0)
    def _init():
        y_ref[pl.ds(row0, _T), :] = jnp.zeros((_T, d), jnp.float32)

    rowid = row0 + lax.broadcasted_iota(jnp.int32, (_T, 1), 0)

    def expert(wg_ref, wu_ref, wd_ref, rmask):
        g = jnp.dot(x, wg_ref[...], preferred_element_type=jnp.float32)
        u = jnp.dot(x, wu_ref[...], preferred_element_type=jnp.float32)
        h = jax.nn.silu(g) * u
        yp = jnp.dot(h, wd_ref[...], preferred_element_type=jnp.float32)
        y_ref[pl.ds(row0, _T), :] += jnp.where(rmask, yp, 0.0)

    @pl.when(row0 < n0)
    def _und():
        expert(wgu, wuu, wdu, rowid < n0)

    @pl.when(row0 + _T > n0)
    def _gen():
        expert(wgg, wug, wdg, rowid >= n0)


def _moe_tc(n0_arr, xp, wg_und, wu_und, wd_und, wg_gen, wu_gen, wd_gen):
    l, d = xp.shape
    f_dim = wg_und.shape[1]
    nf = (f_dim + _FT - 1) // _FT
    nt = l // _T

    w_in_spec = pl.BlockSpec((d, _FT), lambda f, t, n0: (0, f))
    w_dn_spec = pl.BlockSpec((_FT, d), lambda f, t, n0: (f, 0))
    full_spec = pl.BlockSpec((l, d), lambda f, t, n0: (0, 0))

    grid_spec = pltpu.PrefetchScalarGridSpec(
        num_scalar_prefetch=1,
        grid=(nf, nt),
        in_specs=[full_spec, w_in_spec, w_in_spec, w_dn_spec,
                  w_in_spec, w_in_spec, w_dn_spec],
        out_specs=full_spec,
    )
    return pl.pallas_call(
        functools.partial(_moe_body, l=l, d=d, f_dim=f_dim),
        grid_spec=grid_spec,
        out_shape=jax.ShapeDtypeStruct((l, d), jnp.float32),
        compiler_params=pltpu.CompilerParams(
            dimension_semantics=("arbitrary", "arbitrary")),
    )(n0_arr, xp, wg_und, wu_und, wd_und, wg_gen, wu_gen, wd_gen)


def kernel(hidden_states, gen_token_mask, Wg_und, Wu_und, Wd_und, Wg_gen, Wu_gen, Wd_gen):
    b, l, d = hidden_states.shape
    x = hidden_states.reshape(b * l, d)
    m = gen_token_mask.reshape(b * l).astype(jnp.int32)

    n0 = (b * l) - jnp.sum(m)
    perm = jnp.argsort(m, stable=True).astype(jnp.int32)
    rank0 = jnp.cumsum(1 - m) - 1
    rank1 = jnp.cumsum(m) - 1
    loc = jnp.where(m > 0, n0 + rank1, rank0).astype(jnp.int32)

    xp = _sc_gather_rows(x, perm)
    y = _moe_tc(n0.reshape(1).astype(jnp.int32), xp,
                Wg_und, Wu_und, Wd_und, Wg_gen, Wu_gen, Wd_gen)
    out = _sc_gather_rows(y, loc)
    return out.reshape(b, l, d)


# speedup vs baseline: 1.1952x; 1.1952x over previous
import functools

import jax
import jax.numpy as jnp
from jax import lax
from jax.experimental import pallas as pl
from jax.experimental.pallas import tpu as pltpu
from jax.experimental.pallas import tpu_sc as plsc


_T = 256
_FT = 512
_DT = 256


def _sc_gather_rows(table, idx, chunk=16):
    rows, d = idx.shape[0], table.shape[1]
    info = plsc.get_sparse_core_info()
    nw = info.num_cores * info.num_subcores
    per_w = rows // nw
    assert per_w % chunk == 0 and (per_w * (nw - 1)) % 8 == 0
    n_ch = per_w // chunk
    mesh = plsc.VectorSubcoreMesh(core_axis_name="c", subcore_axis_name="s")

    @functools.partial(
        pl.kernel,
        mesh=mesh,
        out_type=jax.ShapeDtypeStruct((rows, d), table.dtype),
        scratch_types=[
            pltpu.VMEM((chunk,), jnp.int32),
            pltpu.VMEM((chunk, d), table.dtype),
            pltpu.SemaphoreType.DMA,
        ],
    )
    def gather_k(table_hbm, idx_hbm, out_hbm, idx_v, rows_v, sem):
        wid = lax.axis_index("s") * info.num_cores + lax.axis_index("c")
        base = wid * per_w
        for c in range(n_ch):
            off = base + c * chunk
            pltpu.sync_copy(idx_hbm.at[pl.ds(off, chunk)], idx_v)
            pltpu.async_copy(table_hbm.at[idx_v], rows_v, sem).wait()
            pltpu.sync_copy(rows_v, out_hbm.at[pl.ds(off, chunk)])

    return gather_k(table, idx)


def _p1_body(n0_ref, x_ref, wgu_f, wuu_f, wgg_f, wug_f, h_ref,
             wgu_b, wuu_b, wgg_b, wug_b):
    t = pl.program_id(1)
    n0 = n0_ref[0]
    row0 = t * _T

    @pl.when(t == 0)
    def _cast():
        wgu_b[...] = wgu_f[...].astype(jnp.bfloat16)
        wuu_b[...] = wuu_f[...].astype(jnp.bfloat16)
        wgg_b[...] = wgg_f[...].astype(jnp.bfloat16)
        wug_b[...] = wug_f[...].astype(jnp.bfloat16)

    x = x_ref[pl.ds(row0, _T), :]

    def mk(wg_b, wu_b):
        g = jnp.dot(x, wg_b[...], preferred_element_type=jnp.float32)
        u = jnp.dot(x, wu_b[...], preferred_element_type=jnp.float32)
        return (jax.nn.silu(g) * u).astype(jnp.bfloat16)

    @pl.when(row0 + _T <= n0)
    def _und():
        h_ref[...] = mk(wgu_b, wuu_b)

    @pl.when(row0 >= n0)
    def _gen():
        h_ref[...] = mk(wgg_b, wug_b)

    @pl.when((row0 < n0) & (row0 + _T > n0))
    def _mix():
        rmask = row0 + lax.broadcasted_iota(jnp.int32, (_T, 1), 0) < n0
        h_ref[...] = jnp.where(rmask, mk(wgu_b, wuu_b), mk(wgg_b, wug_b))


def _p2_body(n0_ref, h_ref, wdu_f, wdg_f, y_ref, wdu_b, wdg_b):
    t = pl.program_id(1)
    n0 = n0_ref[0]
    row0 = t * _T

    @pl.when(t == 0)
    def _cast():
        wdu_b[...] = wdu_f[...].astype(jnp.bfloat16)
        wdg_b[...] = wdg_f[...].astype(jnp.bfloat16)

    h = h_ref[pl.ds(row0, _T), :]

    @pl.when(row0 + _T <= n0)
    def _und():
        y_ref[...] = jnp.dot(h, wdu_b[...], preferred_element_type=jnp.float32)

    @pl.when(row0 >= n0)
    def _gen():
        y_ref[...] = jnp.dot(h, wdg_b[...], preferred_element_type=jnp.float32)

    @pl.when((row0 < n0) & (row0 + _T > n0))
    def _mix():
        rmask = row0 + lax.broadcasted_iota(jnp.int32, (_T, 1), 0) < n0
        y_ref[...] = jnp.where(
            rmask,
            jnp.dot(h, wdu_b[...], preferred_element_type=jnp.float32),
            jnp.dot(h, wdg_b[...], preferred_element_type=jnp.float32))


def _moe_tc(n0_arr, xp_bf, wg_und, wu_und, wd_und, wg_gen, wu_gen, wd_gen):
    l, d = xp_bf.shape
    f_dim = wg_und.shape[1]
    nf = (f_dim + _FT - 1) // _FT
    nt = l // _T
    nd = d // _DT

    h = pl.pallas_call(
        _p1_body,
        grid_spec=pltpu.PrefetchScalarGridSpec(
            num_scalar_prefetch=1,
            grid=(nf, nt),
            in_specs=[
                pl.BlockSpec((l, d), lambda f, t, n0: (0, 0)),
                pl.BlockSpec((d, _FT), lambda f, t, n0: (0, f)),
                pl.BlockSpec((d, _FT), lambda f, t, n0: (0, f)),
                pl.BlockSpec((d, _FT), lambda f, t, n0: (0, f)),
                pl.BlockSpec((d, _FT), lambda f, t, n0: (0, f)),
            ],
            out_specs=pl.BlockSpec((_T, _FT), lambda f, t, n0: (t, f)),
            scratch_shapes=[pltpu.VMEM((d, _FT), jnp.bfloat16)] * 4,
        ),
        out_shape=jax.ShapeDtypeStruct((l, f_dim), jnp.bfloat16),
        compiler_params=pltpu.CompilerParams(
            dimension_semantics=("arbitrary", "arbitrary")),
    )(n0_arr, xp_bf, wg_und, wu_und, wg_gen, wu_gen)

    y = pl.pallas_call(
        _p2_body,
        grid_spec=pltpu.PrefetchScalarGridSpec(
            num_scalar_prefetch=1,
            grid=(nd, nt),
            in_specs=[
                pl.BlockSpec((l, f_dim), lambda dd, t, n0: (0, 0)),
                pl.BlockSpec((f_dim, _DT), lambda dd, t, n0: (0, dd)),
                pl.BlockSpec((f_dim, _DT), lambda dd, t, n0: (0, dd)),
            ],
            out_specs=pl.BlockSpec((_T, _DT), lambda dd, t, n0: (t, dd)),
            scratch_shapes=[pltpu.VMEM((f_dim, _DT), jnp.bfloat16)] * 2,
        ),
        out_shape=jax.ShapeDtypeStruct((l, d), jnp.float32),
        compiler_params=pltpu.CompilerParams(
            dimension_semantics=("arbitrary", "arbitrary")),
    )(n0_arr, h, wd_und, wd_gen)
    return y


def kernel(hidden_states, gen_token_mask, Wg_und, Wu_und, Wd_und, Wg_gen, Wu_gen, Wd_gen):
    b, l, d = hidden_states.shape
    x = hidden_states.reshape(b * l, d)
    m = gen_token_mask.reshape(b * l).astype(jnp.int32)

    n0 = (b * l) - jnp.sum(m)
    perm = jnp.argsort(m, stable=True).astype(jnp.int32)
    rank0 = jnp.cumsum(1 - m) - 1
    rank1 = jnp.cumsum(m) - 1
    loc = jnp.where(m > 0, n0 + rank1, rank0).astype(jnp.int32)

    x_bits = lax.bitcast_convert_type(
        x.astype(jnp.bfloat16).reshape(b * l, d // 2, 2), jnp.int32)
    xp_bits = _sc_gather_rows(x_bits, perm)
    xp_bf = lax.bitcast_convert_type(xp_bits, jnp.bfloat16).reshape(b * l, d)

    y = _moe_tc(n0.reshape(1).astype(jnp.int32), xp_bf,
                Wg_und, Wu_und, Wd_und, Wg_gen, Wu_gen, Wd_gen)
    out = _sc_gather_rows(y, loc)
    return out.reshape(b, l, d)
